# baseline (speedup 1.0000x reference)
import functools

import jax
import jax.numpy as jnp
from jax import lax
from jax.experimental import pallas as pl
from jax.experimental.pallas import tpu as pltpu
from jax.experimental.pallas import tpu_sc as plsc

NPAD = 10240
NWORK = 32
SC_CORES = 2
SC_SUBCORES = 16


def _mm_tables(x, wg, wd, bias, apply_elu, bm=1024):
    np_, k = x.shape
    twg = wg.shape[1]
    twd = wd.shape[1]

    def body(x_ref, wg_ref, wd_ref, b_ref, og_ref, od_ref):
        z = x_ref[...] + b_ref[...]
        if apply_elu:
            z = jnp.where(z > 0, z, jnp.exp(z) - 1.0)
        og_ref[...] = jnp.dot(z, wg_ref[...], preferred_element_type=jnp.float32)
        od_ref[...] = jnp.dot(z, wd_ref[...], preferred_element_type=jnp.float32)

    return pl.pallas_call(
        body,
        grid=(np_ // bm,),
        in_specs=[
            pl.BlockSpec((bm, k), lambda i: (i, 0)),
            pl.BlockSpec((k, twg), lambda i: (0, 0)),
            pl.BlockSpec((k, twd), lambda i: (0, 0)),
            pl.BlockSpec((1, k), lambda i: (0, 0)),
        ],
        out_specs=[
            pl.BlockSpec((bm, twg), lambda i: (i, 0)),
            pl.BlockSpec((bm, twd), lambda i: (i, 0)),
        ],
        out_shape=[
            jax.ShapeDtypeStruct((np_, twg), jnp.float32),
            jax.ShapeDtypeStruct((np_, twd), jnp.float32),
        ],
    )(x, wg, wd, bias)


def _final_out(z3, b3, bm=1024):
    np_, tw = z3.shape

    def body(z_ref, b_ref, o_ref):
        o_ref[...] = z_ref[:, :40] + b_ref[...]

    return pl.pallas_call(
        body,
        grid=(np_ // bm,),
        in_specs=[
            pl.BlockSpec((bm, tw), lambda i: (i, 0)),
            pl.BlockSpec((1, 40), lambda i: (0, 0)),
        ],
        out_specs=pl.BlockSpec((bm, 40), lambda i: (i, 0)),
        out_shape=jax.ShapeDtypeStruct((np_, 40), jnp.float32),
    )(z3, b3)


def _sc_edge(g, ad, src_s, dst_s, eb, z0, *, H, C, TW, CN, CNP, CPW, T):
    F = H * C
    NCB = C // 16
    CREM = C - NCB * 16

    mesh = plsc.VectorSubcoreMesh(
        core_axis_name="c", subcore_axis_name="s",
        num_cores=SC_CORES, num_subcores=SC_SUBCORES)

    @functools.partial(
        pl.kernel,
        out_type=jax.ShapeDtypeStruct((NPAD * TW,), jnp.float32),
        mesh=mesh,
        compiler_params=pltpu.CompilerParams(
            use_tc_tiling_on_sc=False, needs_layout_passes=False),
        scratch_types=[
            pltpu.VMEM((CNP * TW,), jnp.float32),
            pltpu.VMEM((T, TW), jnp.float32),
            pltpu.VMEM((T,), jnp.int32),
            pltpu.VMEM((T,), jnp.int32),
            pltpu.VMEM((CN * 16,), jnp.float32),
            pltpu.VMEM((16,), jnp.float32),
            pltpu.VMEM((16,), jnp.float32),
            pltpu.SemaphoreType.DMA,
        ],
    )
    def k(g_hbm, ad_hbm, src_hbm, dst_hbm, eb_hbm, z0_hbm, z_hbm,
          acc, rows2, src_v, dst_v, ad_v, e0_v, e1_v, sem):
        wid = lax.axis_index("s") * SC_CORES + lax.axis_index("c")
        lane = lax.iota(jnp.int32, 16)

        def chunk(j, _):
            cidx = wid * CPW + j
            n0 = cidx * CN
            pltpu.sync_copy(eb_hbm.at[pl.ds(cidx * 32, 16)], e0_v)
            pltpu.sync_copy(eb_hbm.at[pl.ds(cidx * 32 + 16, 16)], e1_v)
            e0 = jnp.max(e0_v[...]).astype(jnp.int32)
            e1 = jnp.max(e1_v[...]).astype(jnp.int32)
            e0a = (e0 // 8) * 8
            nb = (e1 - e0a + (T - 1)) // T
            pltpu.sync_copy(z0_hbm, acc)
            pltpu.sync_copy(ad_hbm.at[pl.ds(n0 * 16, CN * 16)], ad_v)

            def batch(i, _):
                ebase = e0a + i * T
                pltpu.sync_copy(src_hbm.at[pl.ds(ebase, T)], src_v)
                pltpu.sync_copy(dst_hbm.at[pl.ds(ebase, T)], dst_v)
                pltpu.async_copy(g_hbm.at[src_v], rows2, sem).wait()

                def subgroup(sg, _):
                    ei = sg * 16 + lane
                    eg = ebase + ei
                    valid = (eg >= e0) & (eg < e1)
                    dstv = dst_v[pl.ds(sg * 16, 16)]
                    dl = jnp.clip(dstv - n0, 0, CN - 1)
                    dlt = dl * TW
                    dl16 = dl * 16
                    fcol = jnp.full((16,), F, jnp.int32)

                    exs = []
                    for h in range(H):
                        asrc = plsc.load_gather(rows2, [ei, fcol + h])
                        adst = plsc.load_gather(ad_v, [dl16 + h])
                        al = asrc + adst
                        al = jnp.where(al >= 0, al, 0.2 * al)
                        ex = jnp.where(valid, jnp.exp(al), 0.0)
                        plsc.addupdate_scatter(acc, [dlt + (F + h)], ex)
                        exs.append(ex)

                    def lanebody(l, _):
                        il = jnp.full((16,), 0, jnp.int32) + l
                        dltl = jnp.take_along_axis(dlt, il, axis=0)
                        rowv = jnp.full((16,), 0, jnp.int32) + (sg * 16 + l)
                        for h in range(H):
                            exh = jnp.take_along_axis(exs[h], il, axis=0)
                            cb0 = h * C + lane
                            ab0 = dltl + cb0
                            for cb in range(NCB):
                                cv = plsc.load_gather(
                                    rows2, [rowv, cb0 + cb * 16])
                                plsc.addupdate_scatter(
                                    acc, [ab0 + cb * 16], cv * exh)
                            if CREM:
                                cmsk = lane < CREM
                                cv = plsc.load_gather(
                                    rows2, [rowv, cb0 + NCB * 16], mask=cmsk)
                                plsc.addupdate_scatter(
                                    acc, [ab0 + NCB * 16], cv * exh,
                                    mask=cmsk)
                        return 0

                    lax.fori_loop(0, 16, lanebody, 0)
                    return 0

                lax.fori_loop(0, T // 16, subgroup, 0)
                return 0

            lax.fori_loop(0, nb, batch, 0)

            def rgrp(rg, _):
                rt = (rg * 16 + lane) * TW

                def dhead(h, _):
                    dv = plsc.load_gather(acc, [rt + (F + h)])
                    rv = 1.0 / dv

                    def dcol(cb, _):
                        base = h * C + cb * 16
                        for kk in range(16):
                            v = plsc.load_gather(acc, [rt + (base + kk)])
                            plsc.store_scatter(acc, [rt + (base + kk)], v * rv)
                        return 0

                    lax.fori_loop(0, NCB, dcol, 0)
                    for kk in range(CREM):
                        c = h * C + NCB * 16 + kk
                        v = plsc.load_gather(acc, [rt + c])
                        plsc.store_scatter(acc, [rt + c], v * rv)
                    return 0

                lax.fori_loop(0, H, dhead, 0)
                return 0

            lax.fori_loop(0, CNP // 16, rgrp, 0)
            pltpu.sync_copy(acc.at[pl.ds(0, CN * TW)],
                            z_hbm.at[pl.ds(n0 * TW, CN * TW)])
            return 0

        lax.fori_loop(0, CPW, chunk, 0)

    z = k(g, ad.reshape(-1), src_s, dst_s, eb, z0.reshape(-1))
    return z.reshape(NPAD, TW)


def _bounds16(dst_s, cn, nchunk):
    b = jnp.searchsorted(dst_s, jnp.arange(nchunk + 1, dtype=jnp.int32) * cn)
    b = b.astype(jnp.float32)
    pairs = jnp.stack([b[:-1], b[1:]], axis=1)
    return jnp.broadcast_to(pairs[:, :, None], (nchunk, 2, 16)).reshape(-1)


def kernel(x, edge_index, W1, a_src1, a_dst1, b1,
           W2, a_src2, a_dst2, b2, W3, a_src3, a_dst3, b3):
    n, d_in = x.shape
    e = edge_index.shape[1]

    loops = jnp.arange(n, dtype=jnp.int32)
    src = jnp.concatenate([edge_index[0].astype(jnp.int32), loops])
    dst = jnp.concatenate([edge_index[1].astype(jnp.int32), loops])
    perm = jnp.argsort(dst)
    src_s = src[perm]
    dst_s = dst[perm]
    etot = e + n
    epad = ((etot + 128 + 127) // 128) * 128
    src_p = jnp.concatenate(
        [src_s, jnp.zeros((epad - etot,), jnp.int32)])
    dst_p = jnp.concatenate(
        [dst_s, jnp.full((epad - etot,), NPAD - 1, jnp.int32)])
    eb1 = _bounds16(dst_s, 160, 64)
    eb2 = _bounds16(dst_s, 320, 32)

    xp = jnp.zeros((NPAD, d_in), jnp.float32).at[:n].set(x)

    H1, C1 = 8, 64
    w1r = W1.reshape(d_in, H1, C1)
    wg1 = jnp.concatenate(
        [W1, (w1r * a_src1).sum(-1), jnp.zeros((d_in, 8), jnp.float32)], axis=1)
    wd1 = jnp.concatenate(
        [(w1r * a_dst1).sum(-1), jnp.zeros((d_in, 8), jnp.float32)], axis=1)
    be0 = jnp.zeros((1, d_in), jnp.float32)

    wg2 = jnp.zeros((528, 80), jnp.float32)
    wg2 = wg2.at[:512, :64].set(W2).at[:512, 64].set(W2 @ a_src2[0, 0])
    wd2 = jnp.zeros((528, 16), jnp.float32).at[:512, 0].set(W2 @ a_dst2[0, 0])
    be1 = jnp.concatenate([b1, jnp.zeros((16,), jnp.float32)]).reshape(1, 528)

    wg3 = jnp.zeros((80, 48), jnp.float32)
    wg3 = wg3.at[:64, :40].set(W3).at[:64, 40].set(W3 @ a_src3[0, 0])
    wd3 = jnp.zeros((80, 16), jnp.float32).at[:64, 0].set(W3 @ a_dst3[0, 0])
    be2 = jnp.concatenate([b2, jnp.zeros((16,), jnp.float32)]).reshape(1, 80)

    z0a = jnp.zeros((160, 528), jnp.float32)
    z0b = jnp.zeros((320, 80), jnp.float32)
    z0c = jnp.zeros((320, 48), jnp.float32)

    g1, ad1 = _mm_tables(xp, wg1, wd1, be0, apply_elu=False)
    z1 = _sc_edge(g1, ad1, src_p, dst_p, eb1, z0a,
                  H=8, C=64, TW=528, CN=160, CNP=160, CPW=2, T=64)
    g2, ad2 = _mm_tables(z1, wg2, wd2, be1, apply_elu=True)
    z2 = _sc_edge(g2, ad2, src_p, dst_p, eb2, z0b,
                  H=1, C=64, TW=80, CN=320, CNP=320, CPW=1, T=128)
    g3, ad3 = _mm_tables(z2, wg3, wd3, be2, apply_elu=True)
    z3 = _sc_edge(g3, ad3, src_p, dst_p, eb2, z0c,
                  H=1, C=40, TW=48, CN=320, CNP=320, CPW=1, T=128)

    out = _final_out(z3, b3.reshape(1, 40))
    return out[:n]


# speedup vs baseline: 1.5950x; 1.5950x over previous
import functools

import jax
import jax.numpy as jnp
from jax import lax
from jax.experimental import pallas as pl
from jax.experimental.pallas import tpu as pltpu
from jax.experimental.pallas import tpu_sc as plsc

NPAD = 10240
NWORK = 32
SC_CORES = 2
SC_SUBCORES = 16


def _mm_tables(x, wg, wd, bias, apply_elu, bm=1024):
    np_, k = x.shape
    twg = wg.shape[1]
    twd = wd.shape[1]

    def body(x_ref, wg_ref, wd_ref, b_ref, og_ref, od_ref):
        z = x_ref[...] + b_ref[...]
        if apply_elu:
            z = jnp.where(z > 0, z, jnp.exp(z) - 1.0)
        og_ref[...] = jnp.dot(z, wg_ref[...], preferred_element_type=jnp.float32)
        od_ref[...] = jnp.dot(z, wd_ref[...], preferred_element_type=jnp.float32)

    return pl.pallas_call(
        body,
        grid=(np_ // bm,),
        in_specs=[
            pl.BlockSpec((bm, k), lambda i: (i, 0)),
            pl.BlockSpec((k, twg), lambda i: (0, 0)),
            pl.BlockSpec((k, twd), lambda i: (0, 0)),
            pl.BlockSpec((1, k), lambda i: (0, 0)),
        ],
        out_specs=[
            pl.BlockSpec((bm, twg), lambda i: (i, 0)),
            pl.BlockSpec((bm, twd), lambda i: (i, 0)),
        ],
        out_shape=[
            jax.ShapeDtypeStruct((np_, twg), jnp.float32),
            jax.ShapeDtypeStruct((np_, twd), jnp.float32),
        ],
    )(x, wg, wd, bias)


def _final_out(z3, b3, bm=1024):
    np_, tw = z3.shape

    def body(z_ref, b_ref, o_ref):
        o_ref[...] = z_ref[:, :40] + b_ref[...]

    return pl.pallas_call(
        body,
        grid=(np_ // bm,),
        in_specs=[
            pl.BlockSpec((bm, tw), lambda i: (i, 0)),
            pl.BlockSpec((1, 40), lambda i: (0, 0)),
        ],
        out_specs=pl.BlockSpec((bm, 40), lambda i: (i, 0)),
        out_shape=jax.ShapeDtypeStruct((np_, 40), jnp.float32),
    )(z3, b3)


def _sc_edge(g, ad, src_s, dst_s, eb, z0, *, H, C, TW, CN, CNP, CPW, T):
    F = H * C
    NCB = C // 16
    CREM = C - NCB * 16

    mesh = plsc.VectorSubcoreMesh(
        core_axis_name="c", subcore_axis_name="s",
        num_cores=SC_CORES, num_subcores=SC_SUBCORES)

    @functools.partial(
        pl.kernel,
        out_type=jax.ShapeDtypeStruct((NPAD * TW,), jnp.float32),
        mesh=mesh,
        compiler_params=pltpu.CompilerParams(
            use_tc_tiling_on_sc=False, needs_layout_passes=False),
        scratch_types=[
            pltpu.VMEM((CNP * TW,), jnp.float32),
            pltpu.VMEM((T, TW), jnp.float32),
            pltpu.VMEM((T,), jnp.int32),
            pltpu.VMEM((T,), jnp.int32),
            pltpu.VMEM((CN * 16,), jnp.float32),
            pltpu.VMEM((16,), jnp.float32),
            pltpu.VMEM((16,), jnp.float32),
            pltpu.SemaphoreType.DMA,
        ],
    )
    def k(g_hbm, ad_hbm, src_hbm, dst_hbm, eb_hbm, z0_hbm, z_hbm,
          acc, rows2, src_v, dst_v, ad_v, e0_v, e1_v, sem):
        wid = lax.axis_index("s") * SC_CORES + lax.axis_index("c")
        lane = lax.iota(jnp.int32, 16)

        def chunk(j, _):
            cidx = wid * CPW + j
            n0 = cidx * CN
            pltpu.sync_copy(eb_hbm.at[pl.ds(cidx * 32, 16)], e0_v)
            pltpu.sync_copy(eb_hbm.at[pl.ds(cidx * 32 + 16, 16)], e1_v)
            e0 = jnp.max(e0_v[...]).astype(jnp.int32)
            e1 = jnp.max(e1_v[...]).astype(jnp.int32)
            e0a = (e0 // 8) * 8
            nb = (e1 - e0a + (T - 1)) // T
            pltpu.sync_copy(z0_hbm, acc)
            pltpu.sync_copy(ad_hbm.at[pl.ds(n0 * 16, CN * 16)], ad_v)

            def batch(i, _):
                ebase = e0a + i * T
                pltpu.sync_copy(src_hbm.at[pl.ds(ebase, T)], src_v)
                pltpu.sync_copy(dst_hbm.at[pl.ds(ebase, T)], dst_v)
                pltpu.async_copy(g_hbm.at[src_v], rows2, sem).wait()

                def subgroup(sg, _):
                    ei = sg * 16 + lane
                    eg = ebase + ei
                    valid = (eg >= e0) & (eg < e1)
                    dstv = dst_v[pl.ds(sg * 16, 16)]
                    dl = jnp.clip(dstv - n0, 0, CN - 1)
                    dlt = dl * TW
                    dl16 = dl * 16
                    fcol = jnp.full((16,), F, jnp.int32)

                    exs = []
                    for h in range(H):
                        asrc = plsc.load_gather(rows2, [ei, fcol + h])
                        adst = plsc.load_gather(ad_v, [dl16 + h])
                        al = asrc + adst
                        al = jnp.where(al >= 0, al, 0.2 * al)
                        ex = jnp.where(valid, jnp.exp(al), 0.0)
                        plsc.addupdate_scatter(acc, [dlt + (F + h)], ex)
                        exs.append(ex)

                    def lanebody(l):
                        il = jnp.full((16,), 0, jnp.int32) + l
                        dltl = jnp.take_along_axis(dlt, il, axis=0)
                        rowv = jnp.full((16,), 0, jnp.int32) + (sg * 16 + l)
                        for h in range(H):
                            exh = jnp.take_along_axis(exs[h], il, axis=0)
                            cb0 = h * C + lane
                            ab0 = dltl + cb0
                            for cb in range(NCB):
                                cv = plsc.load_gather(
                                    rows2, [rowv, cb0 + cb * 16])
                                plsc.addupdate_scatter(
                                    acc, [ab0 + cb * 16], cv * exh)
                            if CREM:
                                cmsk = lane < CREM
                                cv = plsc.load_gather(
                                    rows2, [rowv, cb0 + NCB * 16], mask=cmsk)
                                plsc.addupdate_scatter(
                                    acc, [ab0 + NCB * 16], cv * exh,
                                    mask=cmsk)

                    plsc.parallel_loop(0, 16, 1, unroll=2)(lanebody)
                    return 0

                lax.fori_loop(0, T // 16, subgroup, 0)
                return 0

            lax.fori_loop(0, nb, batch, 0)

            def rgrp(rg):
                rt = (rg * 16 + lane) * TW

                def dhead(h, _):
                    dv = plsc.load_gather(acc, [rt + (F + h)])
                    rv = 1.0 / dv

                    def dcol(cb, _):
                        base = h * C + cb * 16
                        for kk in range(16):
                            v = plsc.load_gather(acc, [rt + (base + kk)])
                            plsc.store_scatter(acc, [rt + (base + kk)], v * rv)
                        return 0

                    lax.fori_loop(0, NCB, dcol, 0)
                    for kk in range(CREM):
                        c = h * C + NCB * 16 + kk
                        v = plsc.load_gather(acc, [rt + c])
                        plsc.store_scatter(acc, [rt + c], v * rv)
                    return 0

                lax.fori_loop(0, H, dhead, 0)

            plsc.parallel_loop(0, CNP // 16, 1)(rgrp)
            pltpu.sync_copy(acc.at[pl.ds(0, CN * TW)],
                            z_hbm.at[pl.ds(n0 * TW, CN * TW)])
            return 0

        lax.fori_loop(0, CPW, chunk, 0)

    z = k(g, ad.reshape(-1), src_s, dst_s, eb, z0.reshape(-1))
    return z.reshape(NPAD, TW)


def _bounds16(dst_s, cn, nchunk):
    b = jnp.searchsorted(dst_s, jnp.arange(nchunk + 1, dtype=jnp.int32) * cn)
    b = b.astype(jnp.float32)
    pairs = jnp.stack([b[:-1], b[1:]], axis=1)
    return jnp.broadcast_to(pairs[:, :, None], (nchunk, 2, 16)).reshape(-1)


def kernel(x, edge_index, W1, a_src1, a_dst1, b1,
           W2, a_src2, a_dst2, b2, W3, a_src3, a_dst3, b3):
    n, d_in = x.shape
    e = edge_index.shape[1]

    loops = jnp.arange(n, dtype=jnp.int32)
    src = jnp.concatenate([edge_index[0].astype(jnp.int32), loops])
    dst = jnp.concatenate([edge_index[1].astype(jnp.int32), loops])
    perm = jnp.argsort(dst)
    src_s = src[perm]
    dst_s = dst[perm]
    etot = e + n
    epad = ((etot + 128 + 127) // 128) * 128
    src_p = jnp.concatenate(
        [src_s, jnp.zeros((epad - etot,), jnp.int32)])
    dst_p = jnp.concatenate(
        [dst_s, jnp.full((epad - etot,), NPAD - 1, jnp.int32)])
    eb1 = _bounds16(dst_s, 160, 64)
    eb2 = _bounds16(dst_s, 320, 32)

    xp = jnp.zeros((NPAD, d_in), jnp.float32).at[:n].set(x)

    H1, C1 = 8, 64
    w1r = W1.reshape(d_in, H1, C1)
    wg1 = jnp.concatenate(
        [W1, (w1r * a_src1).sum(-1), jnp.zeros((d_in, 8), jnp.float32)], axis=1)
    wd1 = jnp.concatenate(
        [(w1r * a_dst1).sum(-1), jnp.zeros((d_in, 8), jnp.float32)], axis=1)
    be0 = jnp.zeros((1, d_in), jnp.float32)

    wg2 = jnp.zeros((528, 80), jnp.float32)
    wg2 = wg2.at[:512, :64].set(W2).at[:512, 64].set(W2 @ a_src2[0, 0])
    wd2 = jnp.zeros((528, 16), jnp.float32).at[:512, 0].set(W2 @ a_dst2[0, 0])
    be1 = jnp.concatenate([b1, jnp.zeros((16,), jnp.float32)]).reshape(1, 528)

    wg3 = jnp.zeros((80, 48), jnp.float32)
    wg3 = wg3.at[:64, :40].set(W3).at[:64, 40].set(W3 @ a_src3[0, 0])
    wd3 = jnp.zeros((80, 16), jnp.float32).at[:64, 0].set(W3 @ a_dst3[0, 0])
    be2 = jnp.concatenate([b2, jnp.zeros((16,), jnp.float32)]).reshape(1, 80)

    z0a = jnp.zeros((160, 528), jnp.float32)
    z0b = jnp.zeros((320, 80), jnp.float32)
    z0c = jnp.zeros((320, 48), jnp.float32)

    g1, ad1 = _mm_tables(xp, wg1, wd1, be0, apply_elu=False)
    z1 = _sc_edge(g1, ad1, src_p, dst_p, eb1, z0a,
                  H=8, C=64, TW=528, CN=160, CNP=160, CPW=2, T=64)
    g2, ad2 = _mm_tables(z1, wg2, wd2, be1, apply_elu=True)
    z2 = _sc_edge(g2, ad2, src_p, dst_p, eb2, z0b,
                  H=1, C=64, TW=80, CN=320, CNP=320, CPW=1, T=128)
    g3, ad3 = _mm_tables(z2, wg3, wd3, be2, apply_elu=True)
    z3 = _sc_edge(g3, ad3, src_p, dst_p, eb2, z0c,
                  H=1, C=40, TW=48, CN=320, CNP=320, CPW=1, T=128)

    out = _final_out(z3, b3.reshape(1, 40))
    return out[:n]


# speedup vs baseline: 1.5955x; 1.0003x over previous
import functools

import jax
import jax.numpy as jnp
from jax import lax
from jax.experimental import pallas as pl
from jax.experimental.pallas import tpu as pltpu
from jax.experimental.pallas import tpu_sc as plsc

NPAD = 10240
NWORK = 32
SC_CORES = 2
SC_SUBCORES = 16


def _mm_tables(x, wg, wd, bias, apply_elu, bm=1024):
    np_, k = x.shape
    twg = wg.shape[1]
    twd = wd.shape[1]

    def body(x_ref, wg_ref, wd_ref, b_ref, og_ref, od_ref):
        z = x_ref[...] + b_ref[...]
        if apply_elu:
            z = jnp.where(z > 0, z, jnp.exp(z) - 1.0)
        og_ref[...] = jnp.dot(z, wg_ref[...], preferred_element_type=jnp.float32)
        od_ref[...] = jnp.dot(z, wd_ref[...], preferred_element_type=jnp.float32)

    return pl.pallas_call(
        body,
        grid=(np_ // bm,),
        in_specs=[
            pl.BlockSpec((bm, k), lambda i: (i, 0)),
            pl.BlockSpec((k, twg), lambda i: (0, 0)),
            pl.BlockSpec((k, twd), lambda i: (0, 0)),
            pl.BlockSpec((1, k), lambda i: (0, 0)),
        ],
        out_specs=[
            pl.BlockSpec((bm, twg), lambda i: (i, 0)),
            pl.BlockSpec((bm, twd), lambda i: (i, 0)),
        ],
        out_shape=[
            jax.ShapeDtypeStruct((np_, twg), jnp.float32),
            jax.ShapeDtypeStruct((np_, twd), jnp.float32),
        ],
    )(x, wg, wd, bias)


def _final_out(z3, b3, bm=1024):
    np_, tw = z3.shape

    def body(z_ref, b_ref, o_ref):
        o_ref[...] = z_ref[:, :40] + b_ref[...]

    return pl.pallas_call(
        body,
        grid=(np_ // bm,),
        in_specs=[
            pl.BlockSpec((bm, tw), lambda i: (i, 0)),
            pl.BlockSpec((1, 40), lambda i: (0, 0)),
        ],
        out_specs=pl.BlockSpec((bm, 40), lambda i: (i, 0)),
        out_shape=jax.ShapeDtypeStruct((np_, 40), jnp.float32),
    )(z3, b3)


def _sc_edge(g, ad, src_s, dst_s, eb, z0, *, H, C, TW, CN, CNP, CPW, T):
    F = H * C
    NCB = C // 16
    CREM = C - NCB * 16

    mesh = plsc.VectorSubcoreMesh(
        core_axis_name="c", subcore_axis_name="s",
        num_cores=SC_CORES, num_subcores=SC_SUBCORES)

    @functools.partial(
        pl.kernel,
        out_type=jax.ShapeDtypeStruct((NPAD * TW,), jnp.float32),
        mesh=mesh,
        compiler_params=pltpu.CompilerParams(
            use_tc_tiling_on_sc=False, needs_layout_passes=False),
        scratch_types=[
            pltpu.VMEM((CNP * TW,), jnp.float32),
            pltpu.VMEM((T, TW), jnp.float32),
            pltpu.VMEM((T,), jnp.int32),
            pltpu.VMEM((T,), jnp.int32),
            pltpu.VMEM((CN * 16,), jnp.float32),
            pltpu.VMEM((16,), jnp.float32),
            pltpu.VMEM((16,), jnp.float32),
            pltpu.SemaphoreType.DMA,
        ],
    )
    def k(g_hbm, ad_hbm, src_hbm, dst_hbm, eb_hbm, z0_hbm, z_hbm,
          acc, rows2, src_v, dst_v, ad_v, e0_v, e1_v, sem):
        wid = lax.axis_index("s") * SC_CORES + lax.axis_index("c")
        lane = lax.iota(jnp.int32, 16)

        def chunk(j, _):
            cidx = wid * CPW + j
            n0 = cidx * CN
            pltpu.sync_copy(eb_hbm.at[pl.ds(cidx * 32, 16)], e0_v)
            pltpu.sync_copy(eb_hbm.at[pl.ds(cidx * 32 + 16, 16)], e1_v)
            e0 = jnp.max(e0_v[...]).astype(jnp.int32)
            e1 = jnp.max(e1_v[...]).astype(jnp.int32)
            e0a = (e0 // 8) * 8
            nb = (e1 - e0a + (T - 1)) // T
            pltpu.sync_copy(z0_hbm, acc)
            pltpu.sync_copy(ad_hbm.at[pl.ds(n0 * 16, CN * 16)], ad_v)

            def batch(i, _):
                ebase = e0a + i * T
                pltpu.sync_copy(src_hbm.at[pl.ds(ebase, T)], src_v)
                pltpu.sync_copy(dst_hbm.at[pl.ds(ebase, T)], dst_v)
                pltpu.async_copy(g_hbm.at[src_v], rows2, sem).wait()

                def subgroup(sg):
                    ei = sg * 16 + lane
                    eg = ebase + ei
                    valid = (eg >= e0) & (eg < e1)
                    dstv = dst_v[pl.ds(sg * 16, 16)]
                    dl = jnp.clip(dstv - n0, 0, CN - 1)
                    dlt = dl * TW
                    dl16 = dl * 16
                    fcol = jnp.full((16,), F, jnp.int32)

                    exs = []
                    for h in range(H):
                        asrc = plsc.load_gather(rows2, [ei, fcol + h])
                        adst = plsc.load_gather(ad_v, [dl16 + h])
                        al = asrc + adst
                        al = jnp.where(al >= 0, al, 0.2 * al)
                        ex = jnp.where(valid, jnp.exp(al), 0.0)
                        plsc.addupdate_scatter(acc, [dlt + (F + h)], ex)
                        exs.append(ex)

                    def lanebody(l):
                        il = jnp.full((16,), 0, jnp.int32) + l
                        dltl = jnp.take_along_axis(dlt, il, axis=0)
                        rowv = jnp.full((16,), 0, jnp.int32) + (sg * 16 + l)
                        for h in range(H):
                            exh = jnp.take_along_axis(exs[h], il, axis=0)
                            cb0 = h * C + lane
                            ab0 = dltl + cb0
                            for cb in range(NCB):
                                cv = plsc.load_gather(
                                    rows2, [rowv, cb0 + cb * 16])
                                plsc.addupdate_scatter(
                                    acc, [ab0 + cb * 16], cv * exh)
                            if CREM:
                                cmsk = lane < CREM
                                cv = plsc.load_gather(
                                    rows2, [rowv, cb0 + NCB * 16], mask=cmsk)
                                plsc.addupdate_scatter(
                                    acc, [ab0 + NCB * 16], cv * exh,
                                    mask=cmsk)

                    plsc.parallel_loop(0, 16, 1, unroll=2)(lanebody)

                plsc.parallel_loop(0, T // 16, 1)(subgroup)
                return 0

            lax.fori_loop(0, nb, batch, 0)

            def rgrp(rg):
                rt = (rg * 16 + lane) * TW

                def dhead(h, _):
                    dv = plsc.load_gather(acc, [rt + (F + h)])
                    rv = 1.0 / dv

                    def dcol(cb, _):
                        base = h * C + cb * 16
                        for kk in range(16):
                            v = plsc.load_gather(acc, [rt + (base + kk)])
                            plsc.store_scatter(acc, [rt + (base + kk)], v * rv)
                        return 0

                    lax.fori_loop(0, NCB, dcol, 0)
                    for kk in range(CREM):
                        c = h * C + NCB * 16 + kk
                        v = plsc.load_gather(acc, [rt + c])
                        plsc.store_scatter(acc, [rt + c], v * rv)
                    return 0

                lax.fori_loop(0, H, dhead, 0)

            plsc.parallel_loop(0, CNP // 16, 1)(rgrp)
            pltpu.sync_copy(acc.at[pl.ds(0, CN * TW)],
                            z_hbm.at[pl.ds(n0 * TW, CN * TW)])
            return 0

        lax.fori_loop(0, CPW, chunk, 0)

    z = k(g, ad.reshape(-1), src_s, dst_s, eb, z0.reshape(-1))
    return z.reshape(NPAD, TW)


def _bounds16(dst_s, cn, nchunk):
    b = jnp.searchsorted(dst_s, jnp.arange(nchunk + 1, dtype=jnp.int32) * cn)
    b = b.astype(jnp.float32)
    pairs = jnp.stack([b[:-1], b[1:]], axis=1)
    return jnp.broadcast_to(pairs[:, :, None], (nchunk, 2, 16)).reshape(-1)


def kernel(x, edge_index, W1, a_src1, a_dst1, b1,
           W2, a_src2, a_dst2, b2, W3, a_src3, a_dst3, b3):
    n, d_in = x.shape
    e = edge_index.shape[1]

    loops = jnp.arange(n, dtype=jnp.int32)
    src = jnp.concatenate([edge_index[0].astype(jnp.int32), loops])
    dst = jnp.concatenate([edge_index[1].astype(jnp.int32), loops])
    perm = jnp.argsort(dst)
    src_s = src[perm]
    dst_s = dst[perm]
    etot = e + n
    epad = ((etot + 128 + 127) // 128) * 128
    src_p = jnp.concatenate(
        [src_s, jnp.zeros((epad - etot,), jnp.int32)])
    dst_p = jnp.concatenate(
        [dst_s, jnp.full((epad - etot,), NPAD - 1, jnp.int32)])
    eb1 = _bounds16(dst_s, 160, 64)
    eb2 = _bounds16(dst_s, 320, 32)

    xp = jnp.zeros((NPAD, d_in), jnp.float32).at[:n].set(x)

    H1, C1 = 8, 64
    w1r = W1.reshape(d_in, H1, C1)
    wg1 = jnp.concatenate(
        [W1, (w1r * a_src1).sum(-1), jnp.zeros((d_in, 8), jnp.float32)], axis=1)
    wd1 = jnp.concatenate(
        [(w1r * a_dst1).sum(-1), jnp.zeros((d_in, 8), jnp.float32)], axis=1)
    be0 = jnp.zeros((1, d_in), jnp.float32)

    wg2 = jnp.zeros((528, 80), jnp.float32)
    wg2 = wg2.at[:512, :64].set(W2).at[:512, 64].set(W2 @ a_src2[0, 0])
    wd2 = jnp.zeros((528, 16), jnp.float32).at[:512, 0].set(W2 @ a_dst2[0, 0])
    be1 = jnp.concatenate([b1, jnp.zeros((16,), jnp.float32)]).reshape(1, 528)

    wg3 = jnp.zeros((80, 48), jnp.float32)
    wg3 = wg3.at[:64, :40].set(W3).at[:64, 40].set(W3 @ a_src3[0, 0])
    wd3 = jnp.zeros((80, 16), jnp.float32).at[:64, 0].set(W3 @ a_dst3[0, 0])
    be2 = jnp.concatenate([b2, jnp.zeros((16,), jnp.float32)]).reshape(1, 80)

    z0a = jnp.zeros((160, 528), jnp.float32)
    z0b = jnp.zeros((320, 80), jnp.float32)
    z0c = jnp.zeros((320, 48), jnp.float32)

    g1, ad1 = _mm_tables(xp, wg1, wd1, be0, apply_elu=False)
    z1 = _sc_edge(g1, ad1, src_p, dst_p, eb1, z0a,
                  H=8, C=64, TW=528, CN=160, CNP=160, CPW=2, T=64)
    g2, ad2 = _mm_tables(z1, wg2, wd2, be1, apply_elu=True)
    z2 = _sc_edge(g2, ad2, src_p, dst_p, eb2, z0b,
                  H=1, C=64, TW=80, CN=320, CNP=320, CPW=1, T=128)
    g3, ad3 = _mm_tables(z2, wg3, wd3, be2, apply_elu=True)
    z3 = _sc_edge(g3, ad3, src_p, dst_p, eb2, z0c,
                  H=1, C=40, TW=48, CN=320, CNP=320, CPW=1, T=128)

    out = _final_out(z3, b3.reshape(1, 40))
    return out[:n]
